# baseline (speedup 1.0000x reference)
import functools

import jax
import jax.numpy as jnp
from jax import lax
from jax.experimental import pallas as pl
from jax.experimental.pallas import tpu as pltpu
from jax.experimental.pallas import tpu_sc as plsc

N = 10000
E = 320000
C = 128
OC = 64
NP = 10240
NC = 2
NS = 16
NW = NC * NS
EPW = E // NW
K = 80
NCHUNK = EPW // K
RPT = NP // NS

_f32 = jnp.float32
_i32 = jnp.int32

_MESH = dict(
    mesh=plsc.VectorSubcoreMesh(core_axis_name="c", subcore_axis_name="s",
                                num_cores=NC, num_subcores=NS),
    compiler_params=pltpu.CompilerParams(needs_layout_passes=False),
)


def _wid_tile():
    cid = lax.axis_index("c")
    sid = lax.axis_index("s")
    return sid * NC + cid, sid, cid


EPT2 = E // NS
NCH2 = EPT2 // K

_RSQRT_MAGIC = 0x5F3759DF


@functools.partial(
    pl.kernel,
    out_type=jax.ShapeDtypeStruct((NC, NP), _f32),
    scratch_types=[
        pltpu.VMEM((K,), _i32),
        pltpu.VMEM((K,), _i32),
        pltpu.VMEM((K,), _f32),
        pltpu.VMEM((K,), _f32),
        pltpu.VMEM((K,), _f32),
        pltpu.VMEM((RPT,), _f32),
        pltpu.SemaphoreType.DMA,
        pltpu.SemaphoreType.DMA,
        pltpu.SemaphoreType.DMA,
        pltpu.SemaphoreType.DMA,
        pltpu.VMEM_SHARED((NP,), _f32),
    ],
    **_MESH,
)
def _deg_kernel(dst_hbm, ew_hbm, dinv_out,
                dstv0, dstv1, ewv0, ewv1, onesv, tmpv,
                sem_i0, sem_i1, sem_s0, sem_s1,
                tab_sp):
    wid, tile, cid = _wid_tile()
    dstv = (dstv0, dstv1)
    ewv = (ewv0, ewv1)
    sem_i = (sem_i0, sem_i1)
    sem_s = (sem_s0, sem_s1)
    sl = pl.ds(tile * RPT, RPT)
    zero16 = jnp.zeros((16,), _f32)
    for j in range(RPT // 16):
        tmpv[pl.ds(j * 16, 16)] = zero16
    pltpu.sync_copy(tmpv, tab_sp.at[sl])
    ones16 = jnp.ones((16,), _f32)
    for j in range(K // 16):
        onesv[pl.ds(j * 16, 16)] = ones16
    plsc.subcore_barrier()

    def pipeline(with_ew):
        src_v = ewv if with_ew else (onesv, onesv)

        def issue_idx(b, c):
            base = tile * EPT2 + c * K
            pltpu.async_copy(dst_hbm.at[pl.ds(base, K)], dstv[b], sem_i[b])
            if with_ew:
                pltpu.async_copy(ew_hbm.at[pl.ds(base, K)], ewv[b], sem_i[b])

        def wait_idx(b):
            pltpu.make_async_copy(dst_hbm.at[pl.ds(0, K)], dstv[b],
                                  sem_i[b]).wait()
            if with_ew:
                pltpu.make_async_copy(ew_hbm.at[pl.ds(0, K)], ewv[b],
                                      sem_i[b]).wait()

        def issue_scatter(b):
            pltpu.async_copy(src_v[b], tab_sp.at[dstv[b]], sem_s[b], add=True)

        def wait_scatter(b):
            pltpu.make_async_copy(src_v[b], tab_sp.at[dstv[b]],
                                  sem_s[b]).wait()

        def handle(c, b, first=False, last=False):
            nb = 1 - b
            wait_idx(b)
            issue_scatter(b)
            if not last:
                if not first:
                    wait_scatter(nb)
                issue_idx(nb, c + 1)

        issue_idx(0, 0)
        handle(0, 0, first=True)
        handle(1, 1)

        def pair(i2, carry):
            handle(2 * i2, 0)
            handle(2 * i2 + 1, 1)
            return carry

        lax.fori_loop(1, NCH2 // 2 - 1, pair, 0)
        handle(NCH2 - 2, 0)
        handle(NCH2 - 1, 1, last=True)
        wait_scatter(0)
        wait_scatter(1)

    @pl.when(cid == 0)
    def _():
        pipeline(True)

    @pl.when(cid == 1)
    def _():
        pipeline(False)

    plsc.subcore_barrier()
    pltpu.sync_copy(tab_sp.at[sl], tmpv)
    for j in range(RPT // 16):
        dsl = pl.ds(j * 16, 16)
        d = tmpv[dsl] + 1.0
        y = plsc.bitcast(_RSQRT_MAGIC - (plsc.bitcast(d, _i32) >> 1), _f32)
        for _ in range(3):
            y = y * (1.5 - 0.5 * d * y * y)
        tmpv[dsl] = y
    pltpu.sync_copy(tmpv, dinv_out.at[cid, sl])


def _make_agg(scaled: bool):
    scratch = [
        pltpu.VMEM((K,), _i32),
        pltpu.VMEM((K,), _i32),
        pltpu.VMEM((K,), _i32),
        pltpu.VMEM((K,), _i32),
        pltpu.VMEM((K, C), _f32),
        pltpu.VMEM((K, C), _f32),
        pltpu.SemaphoreType.DMA,
        pltpu.SemaphoreType.DMA,
        pltpu.SemaphoreType.DMA,
        pltpu.SemaphoreType.DMA,
        pltpu.SemaphoreType.DMA,
        pltpu.SemaphoreType.DMA,
        pltpu.VMEM_SHARED((NP, C), _f32),
    ]
    if scaled:
        scratch += [
            pltpu.VMEM((K,), _f32),
            pltpu.VMEM((K,), _f32),
            pltpu.VMEM((NP,), _f32),
        ]

    def body(src_hbm, dst_hbm, *rest):
        if scaled:
            (ew_hbm, dinv_hbm, x_hbm, zeros_hbm, z_out,
             srcv0, srcv1, dstv0, dstv1, rows0, rows1,
             sem_i0, sem_i1, sem_g0, sem_g1, sem_s0, sem_s1,
             accum, ewv0, ewv1, dinvt) = rest
            ewv = (ewv0, ewv1)
        else:
            (x_hbm, zeros_hbm, z_out,
             srcv0, srcv1, dstv0, dstv1, rows0, rows1,
             sem_i0, sem_i1, sem_g0, sem_g1, sem_s0, sem_s1,
             accum) = rest
        srcv = (srcv0, srcv1)
        dstv = (dstv0, dstv1)
        rows = (rows0, rows1)
        sem_i = (sem_i0, sem_i1)
        sem_g = (sem_g0, sem_g1)
        sem_s = (sem_s0, sem_s1)
        wid, tile, cid = _wid_tile()
        pltpu.sync_copy(zeros_hbm.at[pl.ds(tile * RPT, RPT)],
                        accum.at[pl.ds(tile * RPT, RPT)])
        if scaled:
            pltpu.sync_copy(dinv_hbm, dinvt)
        plsc.subcore_barrier()

        def issue_idx(b, c):
            base = wid * EPW + c * K
            pltpu.async_copy(src_hbm.at[pl.ds(base, K)], srcv[b], sem_i[b])
            pltpu.async_copy(dst_hbm.at[pl.ds(base, K)], dstv[b], sem_i[b])
            if scaled:
                pltpu.async_copy(ew_hbm.at[pl.ds(base, K)], ewv[b], sem_i[b])

        def wait_idx(b):
            pltpu.make_async_copy(src_hbm.at[pl.ds(0, K)], srcv[b],
                                  sem_i[b]).wait()
            pltpu.make_async_copy(dst_hbm.at[pl.ds(0, K)], dstv[b],
                                  sem_i[b]).wait()
            if scaled:
                pltpu.make_async_copy(ew_hbm.at[pl.ds(0, K)], ewv[b],
                                      sem_i[b]).wait()

        def issue_gather(b):
            pltpu.async_copy(x_hbm.at[srcv[b]], rows[b], sem_g[b])

        def wait_gather(b):
            pltpu.make_async_copy(x_hbm.at[srcv[b]], rows[b], sem_g[b]).wait()

        def issue_scatter(b):
            pltpu.async_copy(rows[b], accum.at[dstv[b]], sem_s[b], add=True)

        def wait_scatter(b):
            pltpu.make_async_copy(rows[b], accum.at[dstv[b]], sem_s[b]).wait()

        def scale(b):
            for j in range(K // 16):
                s16 = srcv[b][pl.ds(j * 16, 16)]
                d16 = dstv[b][pl.ds(j * 16, 16)]
                e16 = ewv[b][pl.ds(j * 16, 16)]
                n16 = (plsc.load_gather(dinvt, [s16]) * e16 *
                       plsc.load_gather(dinvt, [d16]))
                for l in range(16):
                    e = j * 16 + l
                    s = n16[l]
                    for cc in range(C // 16):
                        sl = pl.ds(cc * 16, 16)
                        rows[b][e, sl] = rows[b][e, sl] * s

        def handle(c, b, first=False, last=False):
            nb = 1 - b
            if not last:
                if not first:
                    wait_scatter(nb)
                issue_idx(nb, c + 1)
                wait_idx(nb)
                issue_gather(nb)
            wait_gather(b)
            if scaled:
                scale(b)
            issue_scatter(b)

        issue_idx(0, 0)
        wait_idx(0)
        issue_gather(0)
        handle(0, 0, first=True)
        handle(1, 1)

        def pair(i2, carry):
            handle(2 * i2, 0)
            handle(2 * i2 + 1, 1)
            return carry

        lax.fori_loop(1, NCHUNK // 2, pair, 0)
        handle(NCHUNK - 1, 0, last=True)
        wait_scatter(1)
        wait_scatter(0)
        plsc.subcore_barrier()
        pltpu.sync_copy(accum.at[pl.ds(tile * RPT, RPT)],
                        z_out.at[cid, pl.ds(tile * RPT, RPT)])

    return pl.kernel(
        body,
        out_type=jax.ShapeDtypeStruct((NC, NP, C), _f32),
        scratch_types=scratch,
        **_MESH,
    )


_agg_scaled = _make_agg(True)
_agg_plain = _make_agg(False)


_RB = 2000


def _h_body(z_ref, x_ref, dw_ref, d1_ref, w1_ref, b1_ref, h_ref, y2_ref):
    dw = dw_ref[...]
    s1 = z_ref[0] + z_ref[1] + dw * dw * x_ref[...]
    h = jnp.dot(s1, w1_ref[...], preferred_element_type=_f32) + b1_ref[...]
    h = jnp.maximum(h, 0.0)
    h_ref[...] = h
    y2_ref[...] = d1_ref[...] * h


_h_call = pl.pallas_call(
    _h_body,
    grid=(N // _RB,),
    in_specs=[
        pl.BlockSpec((NC, _RB, C), lambda i: (0, i, 0)),
        pl.BlockSpec((_RB, C), lambda i: (i, 0)),
        pl.BlockSpec((_RB, 1), lambda i: (i, 0)),
        pl.BlockSpec((_RB, 1), lambda i: (i, 0)),
        pl.BlockSpec((C, C), lambda i: (0, 0)),
        pl.BlockSpec((1, C), lambda i: (0, 0)),
    ],
    out_specs=[
        pl.BlockSpec((_RB, C), lambda i: (i, 0)),
        pl.BlockSpec((_RB, C), lambda i: (i, 0)),
    ],
    out_shape=[
        jax.ShapeDtypeStruct((N, C), _f32),
        jax.ShapeDtypeStruct((N, C), _f32),
    ],
)


def _out_body(z_ref, h_ref, d1_ref, wmu_ref, bmu_ref, wls_ref, bls_ref,
              mu_ref, ls_ref):
    d1 = d1_ref[...]
    g = d1 * (z_ref[0] + z_ref[1]) + d1 * d1 * h_ref[...]
    mu_ref[...] = jnp.dot(g, wmu_ref[...], preferred_element_type=_f32) + bmu_ref[...]
    ls_ref[...] = jnp.dot(g, wls_ref[...], preferred_element_type=_f32) + bls_ref[...]


_out_call = pl.pallas_call(
    _out_body,
    grid=(N // _RB,),
    in_specs=[
        pl.BlockSpec((NC, _RB, C), lambda i: (0, i, 0)),
        pl.BlockSpec((_RB, C), lambda i: (i, 0)),
        pl.BlockSpec((_RB, 1), lambda i: (i, 0)),
        pl.BlockSpec((C, OC), lambda i: (0, 0)),
        pl.BlockSpec((1, OC), lambda i: (0, 0)),
        pl.BlockSpec((C, OC), lambda i: (0, 0)),
        pl.BlockSpec((1, OC), lambda i: (0, 0)),
    ],
    out_specs=[
        pl.BlockSpec((_RB, OC), lambda i: (i, 0)),
        pl.BlockSpec((_RB, OC), lambda i: (i, 0)),
    ],
    out_shape=[
        jax.ShapeDtypeStruct((N, OC), _f32),
        jax.ShapeDtypeStruct((N, OC), _f32),
    ],
)


def kernel(X, edge_index, edge_weight, W1, b1, Wmu, bmu, Wls, bls):
    src = edge_index[0].astype(_i32)
    dst = edge_index[1].astype(_i32)
    ew = edge_weight.astype(_f32)
    zeros128 = jnp.zeros((NP, C), _f32)

    dinv = _deg_kernel(dst, ew)
    dinvw, dinv1 = dinv[0], dinv[1]
    dinvw_col = dinvw.reshape(NP, 1)
    dinv1_col = dinv1.reshape(NP, 1)
    z1 = _agg_scaled(src, dst, ew, dinvw, X, zeros128)
    h, y2 = _h_call(z1, X, dinvw_col, dinv1_col, W1, b1.reshape(1, C))
    z2 = _agg_plain(src, dst, y2, zeros128)
    mu, ls = _out_call(z2, h, dinv1_col, Wmu, bmu.reshape(1, OC),
                       Wls, bls.reshape(1, OC))
    return (mu, ls)


# speedup vs baseline: 1.2702x; 1.2702x over previous
import functools

import jax
import jax.numpy as jnp
from jax import lax
from jax.experimental import pallas as pl
from jax.experimental.pallas import tpu as pltpu
from jax.experimental.pallas import tpu_sc as plsc

N = 10000
E = 320000
C = 128
OC = 64
NP = 10240
NC = 2
NS = 16
NW = NC * NS
EPW = E // NW
K = 80
NCHUNK = EPW // K
RPT = NP // NS

_f32 = jnp.float32
_i32 = jnp.int32

_MESH = dict(
    mesh=plsc.VectorSubcoreMesh(core_axis_name="c", subcore_axis_name="s",
                                num_cores=NC, num_subcores=NS),
    compiler_params=pltpu.CompilerParams(needs_layout_passes=False),
)


def _wid_tile():
    cid = lax.axis_index("c")
    sid = lax.axis_index("s")
    return sid * NC + cid, sid, cid


@functools.partial(
    pl.kernel,
    out_type=jax.ShapeDtypeStruct((NC, 2, NP), _f32),
    scratch_types=[
        pltpu.VMEM((K,), _i32),
        pltpu.VMEM((K,), _i32),
        pltpu.VMEM((K,), _f32),
        pltpu.VMEM((K,), _f32),
        pltpu.VMEM((K,), _f32),
        pltpu.VMEM((RPT,), _f32),
        pltpu.SemaphoreType.DMA,
        pltpu.SemaphoreType.DMA,
        pltpu.SemaphoreType.DMA,
        pltpu.SemaphoreType.DMA,
        pltpu.VMEM_SHARED((NP,), _f32),
        pltpu.VMEM_SHARED((NP,), _f32),
    ],
    **_MESH,
)
def _deg_kernel(dst_hbm, ew_hbm, deg_out,
                dstv0, dstv1, ewv0, ewv1, onesv, tmpv,
                sem_i0, sem_i1, sem_s0, sem_s1,
                degw_sp, deg1_sp):
    wid, tile, cid = _wid_tile()
    dstv = (dstv0, dstv1)
    ewv = (ewv0, ewv1)
    sem_i = (sem_i0, sem_i1)
    sem_s = (sem_s0, sem_s1)
    sl = pl.ds(tile * RPT, RPT)
    zero16 = jnp.zeros((16,), _f32)
    for j in range(RPT // 16):
        tmpv[pl.ds(j * 16, 16)] = zero16
    pltpu.sync_copy(tmpv, degw_sp.at[sl])
    pltpu.sync_copy(tmpv, deg1_sp.at[sl])
    ones16 = jnp.ones((16,), _f32)
    for j in range(K // 16):
        onesv[pl.ds(j * 16, 16)] = ones16
    plsc.subcore_barrier()

    def issue_idx(b, c):
        base = wid * EPW + c * K
        pltpu.async_copy(dst_hbm.at[pl.ds(base, K)], dstv[b], sem_i[b])
        pltpu.async_copy(ew_hbm.at[pl.ds(base, K)], ewv[b], sem_i[b])

    def wait_idx(b):
        pltpu.make_async_copy(dst_hbm.at[pl.ds(0, K)], dstv[b], sem_i[b]).wait()
        pltpu.make_async_copy(ew_hbm.at[pl.ds(0, K)], ewv[b], sem_i[b]).wait()

    def issue_scatter(b):
        pltpu.async_copy(ewv[b], degw_sp.at[dstv[b]], sem_s[b], add=True)
        pltpu.async_copy(onesv, deg1_sp.at[dstv[b]], sem_s[b], add=True)

    def wait_scatter(b):
        pltpu.make_async_copy(ewv[b], degw_sp.at[dstv[b]], sem_s[b]).wait()
        pltpu.make_async_copy(onesv, deg1_sp.at[dstv[b]], sem_s[b]).wait()

    def handle(c, b, first=False, last=False):
        nb = 1 - b
        wait_idx(b)
        issue_scatter(b)
        if not last:
            if not first:
                wait_scatter(nb)
            issue_idx(nb, c + 1)

    issue_idx(0, 0)
    handle(0, 0, first=True)
    handle(1, 1)

    def pair(i2, carry):
        handle(2 * i2, 0)
        handle(2 * i2 + 1, 1)
        return carry

    lax.fori_loop(1, NCHUNK // 2, pair, 0)
    handle(NCHUNK - 1, 0, last=True)
    wait_scatter(1)
    wait_scatter(0)
    plsc.subcore_barrier()
    pltpu.sync_copy(degw_sp.at[sl], deg_out.at[cid, 0, sl])
    pltpu.sync_copy(deg1_sp.at[sl], deg_out.at[cid, 1, sl])


def _make_agg(scaled: bool):
    scratch = (
        [pltpu.VMEM((K,), _i32)] * 3 +
        [pltpu.VMEM((K,), _i32)] * 3 +
        [pltpu.VMEM((K, C), _f32)] * 3 +
        [pltpu.SemaphoreType.DMA] * 9 +
        [pltpu.VMEM_SHARED((NP, C), _f32)]
    )
    if scaled:
        scratch += [
            pltpu.VMEM((K,), _f32),
            pltpu.VMEM((K,), _f32),
            pltpu.VMEM((K,), _f32),
            pltpu.VMEM((NP,), _f32),
        ]

    def body(src_hbm, dst_hbm, *rest):
        if scaled:
            (ew_hbm, dinv_hbm, x_hbm, zeros_hbm, z_out,
             srcv0, srcv1, srcv2, dstv0, dstv1, dstv2, rows0, rows1, rows2,
             sem_i0, sem_i1, sem_i2, sem_g0, sem_g1, sem_g2,
             sem_s0, sem_s1, sem_s2,
             accum, ewv0, ewv1, ewv2, dinvt) = rest
            ewv = (ewv0, ewv1, ewv2)
        else:
            (x_hbm, zeros_hbm, z_out,
             srcv0, srcv1, srcv2, dstv0, dstv1, dstv2, rows0, rows1, rows2,
             sem_i0, sem_i1, sem_i2, sem_g0, sem_g1, sem_g2,
             sem_s0, sem_s1, sem_s2,
             accum) = rest
        srcv = (srcv0, srcv1, srcv2)
        dstv = (dstv0, dstv1, dstv2)
        rows = (rows0, rows1, rows2)
        sem_i = (sem_i0, sem_i1, sem_i2)
        sem_g = (sem_g0, sem_g1, sem_g2)
        sem_s = (sem_s0, sem_s1, sem_s2)
        wid, tile, cid = _wid_tile()
        pltpu.sync_copy(zeros_hbm.at[pl.ds(tile * RPT, RPT)],
                        accum.at[pl.ds(tile * RPT, RPT)])
        if scaled:
            pltpu.sync_copy(dinv_hbm, dinvt)
        plsc.subcore_barrier()

        def issue_idx(b, c):
            base = wid * EPW + c * K
            pltpu.async_copy(src_hbm.at[pl.ds(base, K)], srcv[b], sem_i[b])
            pltpu.async_copy(dst_hbm.at[pl.ds(base, K)], dstv[b], sem_i[b])
            if scaled:
                pltpu.async_copy(ew_hbm.at[pl.ds(base, K)], ewv[b], sem_i[b])

        def wait_idx(b):
            pltpu.make_async_copy(src_hbm.at[pl.ds(0, K)], srcv[b],
                                  sem_i[b]).wait()
            pltpu.make_async_copy(dst_hbm.at[pl.ds(0, K)], dstv[b],
                                  sem_i[b]).wait()
            if scaled:
                pltpu.make_async_copy(ew_hbm.at[pl.ds(0, K)], ewv[b],
                                      sem_i[b]).wait()

        def issue_gather(b):
            pltpu.async_copy(x_hbm.at[srcv[b]], rows[b], sem_g[b])

        def wait_gather(b):
            pltpu.make_async_copy(x_hbm.at[srcv[b]], rows[b], sem_g[b]).wait()

        def issue_scatter(b):
            pltpu.async_copy(rows[b], accum.at[dstv[b]], sem_s[b], add=True)

        def wait_scatter(b):
            pltpu.make_async_copy(rows[b], accum.at[dstv[b]], sem_s[b]).wait()

        def scale(b):
            for j in range(K // 16):
                s16 = srcv[b][pl.ds(j * 16, 16)]
                d16 = dstv[b][pl.ds(j * 16, 16)]
                e16 = ewv[b][pl.ds(j * 16, 16)]
                n16 = (plsc.load_gather(dinvt, [s16]) * e16 *
                       plsc.load_gather(dinvt, [d16]))
                for l in range(16):
                    e = j * 16 + l
                    s = n16[l]
                    for cc in range(C // 16):
                        sl = pl.ds(cc * 16, 16)
                        rows[b][e, sl] = rows[b][e, sl] * s

        def handle(c, s0, s1, s2, skip_ws=False, prep_idx=True,
                   prep_gather=True):
            if prep_idx:
                if not skip_ws:
                    wait_scatter(s2)
                issue_idx(s2, c + 2)
            if prep_gather:
                wait_idx(s1)
                issue_gather(s1)
            wait_gather(s0)
            if scaled:
                scale(s0)
            issue_scatter(s0)

        issue_idx(0, 0)
        issue_idx(1, 1)
        wait_idx(0)
        issue_gather(0)
        handle(0, 0, 1, 2, skip_ws=True)
        handle(1, 1, 2, 0)
        handle(2, 2, 0, 1)

        def triple(i3, carry):
            handle(3 * i3, 0, 1, 2)
            handle(3 * i3 + 1, 1, 2, 0)
            handle(3 * i3 + 2, 2, 0, 1)
            return carry

        lax.fori_loop(1, (NCHUNK - 2) // 3, triple, 0)
        handle(NCHUNK - 2, 0, 1, 2, prep_idx=False)
        handle(NCHUNK - 1, 1, 2, 0, prep_idx=False, prep_gather=False)
        wait_scatter(2)
        wait_scatter(0)
        wait_scatter(1)
        plsc.subcore_barrier()
        pltpu.sync_copy(accum.at[pl.ds(tile * RPT, RPT)],
                        z_out.at[cid, pl.ds(tile * RPT, RPT)])

    return pl.kernel(
        body,
        out_type=jax.ShapeDtypeStruct((NC, NP, C), _f32),
        scratch_types=scratch,
        **_MESH,
    )


_agg_scaled = _make_agg(True)
_agg_plain = _make_agg(False)


def _dinv_body(deg_ref, dinvw_ref, dinv1_ref):
    d = deg_ref[0] + deg_ref[1]
    dinvw_ref[...] = lax.rsqrt(d[0] + 1.0)
    dinv1_ref[...] = lax.rsqrt(d[1] + 1.0)


_dinv_call = pl.pallas_call(
    _dinv_body,
    out_shape=[
        jax.ShapeDtypeStruct((NP,), _f32),
        jax.ShapeDtypeStruct((NP,), _f32),
    ],
)

_RB = 2000


def _h_body(z_ref, x_ref, dw_ref, d1_ref, w1_ref, b1_ref, h_ref, y2_ref):
    dw = dw_ref[...]
    s1 = z_ref[0] + z_ref[1] + dw * dw * x_ref[...]
    h = jnp.dot(s1, w1_ref[...], preferred_element_type=_f32) + b1_ref[...]
    h = jnp.maximum(h, 0.0)
    h_ref[...] = h
    y2_ref[...] = d1_ref[...] * h


_h_call = pl.pallas_call(
    _h_body,
    grid=(N // _RB,),
    in_specs=[
        pl.BlockSpec((NC, _RB, C), lambda i: (0, i, 0)),
        pl.BlockSpec((_RB, C), lambda i: (i, 0)),
        pl.BlockSpec((_RB, 1), lambda i: (i, 0)),
        pl.BlockSpec((_RB, 1), lambda i: (i, 0)),
        pl.BlockSpec((C, C), lambda i: (0, 0)),
        pl.BlockSpec((1, C), lambda i: (0, 0)),
    ],
    out_specs=[
        pl.BlockSpec((_RB, C), lambda i: (i, 0)),
        pl.BlockSpec((_RB, C), lambda i: (i, 0)),
    ],
    out_shape=[
        jax.ShapeDtypeStruct((N, C), _f32),
        jax.ShapeDtypeStruct((N, C), _f32),
    ],
)


def _out_body(z_ref, h_ref, d1_ref, wmu_ref, bmu_ref, wls_ref, bls_ref,
              mu_ref, ls_ref):
    d1 = d1_ref[...]
    g = d1 * (z_ref[0] + z_ref[1]) + d1 * d1 * h_ref[...]
    mu_ref[...] = jnp.dot(g, wmu_ref[...], preferred_element_type=_f32) + bmu_ref[...]
    ls_ref[...] = jnp.dot(g, wls_ref[...], preferred_element_type=_f32) + bls_ref[...]


_out_call = pl.pallas_call(
    _out_body,
    grid=(N // _RB,),
    in_specs=[
        pl.BlockSpec((NC, _RB, C), lambda i: (0, i, 0)),
        pl.BlockSpec((_RB, C), lambda i: (i, 0)),
        pl.BlockSpec((_RB, 1), lambda i: (i, 0)),
        pl.BlockSpec((C, OC), lambda i: (0, 0)),
        pl.BlockSpec((1, OC), lambda i: (0, 0)),
        pl.BlockSpec((C, OC), lambda i: (0, 0)),
        pl.BlockSpec((1, OC), lambda i: (0, 0)),
    ],
    out_specs=[
        pl.BlockSpec((_RB, OC), lambda i: (i, 0)),
        pl.BlockSpec((_RB, OC), lambda i: (i, 0)),
    ],
    out_shape=[
        jax.ShapeDtypeStruct((N, OC), _f32),
        jax.ShapeDtypeStruct((N, OC), _f32),
    ],
)


def kernel(X, edge_index, edge_weight, W1, b1, Wmu, bmu, Wls, bls):
    src = edge_index[0].astype(_i32)
    dst = edge_index[1].astype(_i32)
    ew = edge_weight.astype(_f32)
    zeros128 = jnp.zeros((NP, C), _f32)

    deg = _deg_kernel(dst, ew)
    dinvw, dinv1 = _dinv_call(deg)
    dinvw_col = dinvw.reshape(NP, 1)
    dinv1_col = dinv1.reshape(NP, 1)
    z1 = _agg_scaled(src, dst, ew, dinvw, X, zeros128)
    h, y2 = _h_call(z1, X, dinvw_col, dinv1_col, W1, b1.reshape(1, C))
    z2 = _agg_plain(src, dst, y2, zeros128)
    mu, ls = _out_call(z2, h, dinv1_col, Wmu, bmu.reshape(1, OC),
                       Wls, bls.reshape(1, OC))
    return (mu, ls)


# speedup vs baseline: 1.3393x; 1.0544x over previous
import functools

import jax
import jax.numpy as jnp
from jax import lax
from jax.experimental import pallas as pl
from jax.experimental.pallas import tpu as pltpu
from jax.experimental.pallas import tpu_sc as plsc

N = 10000
E = 320000
C = 128
OC = 64
NP = 10240
NC = 2
NS = 16
NW = NC * NS
EPW = E // NW
K = 80
NCHUNK = EPW // K
RPT = NP // NS

_f32 = jnp.float32
_i32 = jnp.int32

_MESH = dict(
    mesh=plsc.VectorSubcoreMesh(core_axis_name="c", subcore_axis_name="s",
                                num_cores=NC, num_subcores=NS),
    compiler_params=pltpu.CompilerParams(needs_layout_passes=False),
)


def _wid_tile():
    cid = lax.axis_index("c")
    sid = lax.axis_index("s")
    return sid * NC + cid, sid, cid


@functools.partial(
    pl.kernel,
    out_type=jax.ShapeDtypeStruct((NC, 2, NP), _f32),
    scratch_types=(
        [pltpu.VMEM((K,), _i32)] * 3 +
        [pltpu.VMEM((K,), _f32)] * 3 +
        [pltpu.VMEM((K,), _f32), pltpu.VMEM((RPT,), _f32)] +
        [pltpu.SemaphoreType.DMA] * 6 +
        [pltpu.VMEM_SHARED((NP,), _f32)] * 2
    ),
    **_MESH,
)
def _deg_kernel(dst_hbm, ew_hbm, deg_out,
                dstv0, dstv1, dstv2, ewv0, ewv1, ewv2, onesv, tmpv,
                sem_i0, sem_i1, sem_i2, sem_s0, sem_s1, sem_s2,
                degw_sp, deg1_sp):
    wid, tile, cid = _wid_tile()
    dstv = (dstv0, dstv1, dstv2)
    ewv = (ewv0, ewv1, ewv2)
    sem_i = (sem_i0, sem_i1, sem_i2)
    sem_s = (sem_s0, sem_s1, sem_s2)
    sl = pl.ds(tile * RPT, RPT)
    zero16 = jnp.zeros((16,), _f32)
    for j in range(RPT // 16):
        tmpv[pl.ds(j * 16, 16)] = zero16
    pltpu.sync_copy(tmpv, degw_sp.at[sl])
    pltpu.sync_copy(tmpv, deg1_sp.at[sl])
    ones16 = jnp.ones((16,), _f32)
    for j in range(K // 16):
        onesv[pl.ds(j * 16, 16)] = ones16
    plsc.subcore_barrier()

    def issue_idx(b, c):
        base = wid * EPW + c * K
        pltpu.async_copy(dst_hbm.at[pl.ds(base, K)], dstv[b], sem_i[b])
        pltpu.async_copy(ew_hbm.at[pl.ds(base, K)], ewv[b], sem_i[b])

    def wait_idx(b):
        pltpu.make_async_copy(dst_hbm.at[pl.ds(0, K)], dstv[b], sem_i[b]).wait()
        pltpu.make_async_copy(ew_hbm.at[pl.ds(0, K)], ewv[b], sem_i[b]).wait()

    def issue_scatter(b):
        pltpu.async_copy(ewv[b], degw_sp.at[dstv[b]], sem_s[b], add=True)
        pltpu.async_copy(onesv, deg1_sp.at[dstv[b]], sem_s[b], add=True)

    def wait_scatter(b):
        pltpu.make_async_copy(ewv[b], degw_sp.at[dstv[b]], sem_s[b]).wait()
        pltpu.make_async_copy(onesv, deg1_sp.at[dstv[b]], sem_s[b]).wait()

    def handle(c, s0, s1, s2, skip_ws=False, prep=True):
        wait_idx(s0)
        issue_scatter(s0)
        if prep:
            if not skip_ws:
                wait_scatter(s2)
            issue_idx(s2, c + 2)

    issue_idx(0, 0)
    issue_idx(1, 1)
    handle(0, 0, 1, 2, skip_ws=True)
    handle(1, 1, 2, 0)
    handle(2, 2, 0, 1)

    def triple(i3, carry):
        handle(3 * i3, 0, 1, 2)
        handle(3 * i3 + 1, 1, 2, 0)
        handle(3 * i3 + 2, 2, 0, 1)
        return carry

    lax.fori_loop(1, (NCHUNK - 2) // 3, triple, 0)
    handle(NCHUNK - 2, 0, 1, 2, prep=False)
    handle(NCHUNK - 1, 1, 2, 0, prep=False)
    wait_scatter(2)
    wait_scatter(0)
    wait_scatter(1)
    plsc.subcore_barrier()
    pltpu.sync_copy(degw_sp.at[sl], deg_out.at[cid, 0, sl])
    pltpu.sync_copy(deg1_sp.at[sl], deg_out.at[cid, 1, sl])


def _make_agg(scaled: bool):
    scratch = (
        [pltpu.VMEM((K,), _i32)] * 3 +
        [pltpu.VMEM((K,), _i32)] * 3 +
        [pltpu.VMEM((K, C), _f32)] * 3 +
        [pltpu.SemaphoreType.DMA] * 9 +
        [pltpu.VMEM_SHARED((NP, C), _f32)]
    )
    if scaled:
        scratch += [
            pltpu.VMEM((K,), _f32),
            pltpu.VMEM((K,), _f32),
            pltpu.VMEM((K,), _f32),
            pltpu.VMEM((NP,), _f32),
            pltpu.VMEM((RPT,), _f32),
            pltpu.VMEM((RPT,), _f32),
            pltpu.VMEM_SHARED((NP,), _f32),
        ]

    def body(src_hbm, dst_hbm, *rest):
        if scaled:
            (ew_hbm, deg_hbm, x_hbm, zeros_hbm, z_out,
             srcv0, srcv1, srcv2, dstv0, dstv1, dstv2, rows0, rows1, rows2,
             sem_i0, sem_i1, sem_i2, sem_g0, sem_g1, sem_g2,
             sem_s0, sem_s1, sem_s2,
             accum, ewv0, ewv1, ewv2, dinvt, pa, pb, dinv_sp) = rest
            ewv = (ewv0, ewv1, ewv2)
        else:
            (x_hbm, zeros_hbm, z_out,
             srcv0, srcv1, srcv2, dstv0, dstv1, dstv2, rows0, rows1, rows2,
             sem_i0, sem_i1, sem_i2, sem_g0, sem_g1, sem_g2,
             sem_s0, sem_s1, sem_s2,
             accum) = rest
        srcv = (srcv0, srcv1, srcv2)
        dstv = (dstv0, dstv1, dstv2)
        rows = (rows0, rows1, rows2)
        sem_i = (sem_i0, sem_i1, sem_i2)
        sem_g = (sem_g0, sem_g1, sem_g2)
        sem_s = (sem_s0, sem_s1, sem_s2)
        wid, tile, cid = _wid_tile()
        sl = pl.ds(tile * RPT, RPT)
        pltpu.sync_copy(zeros_hbm.at[sl], accum.at[sl])
        if scaled:
            pltpu.sync_copy(deg_hbm.at[0, 0, sl], pa)
            pltpu.sync_copy(deg_hbm.at[1, 0, sl], pb)
            for j in range(RPT // 16):
                dsl = pl.ds(j * 16, 16)
                d = pa[dsl] + pb[dsl] + 1.0
                y = plsc.bitcast(0x5F3759DF - (plsc.bitcast(d, _i32) >> 1),
                                 _f32)
                for _ in range(3):
                    y = y * (1.5 - 0.5 * d * y * y)
                pa[dsl] = y
            pltpu.sync_copy(pa, dinv_sp.at[sl])
        plsc.subcore_barrier()
        if scaled:
            pltpu.sync_copy(dinv_sp, dinvt)

        def issue_idx(b, c):
            base = wid * EPW + c * K
            pltpu.async_copy(src_hbm.at[pl.ds(base, K)], srcv[b], sem_i[b])
            pltpu.async_copy(dst_hbm.at[pl.ds(base, K)], dstv[b], sem_i[b])
            if scaled:
                pltpu.async_copy(ew_hbm.at[pl.ds(base, K)], ewv[b], sem_i[b])

        def wait_idx(b):
            pltpu.make_async_copy(src_hbm.at[pl.ds(0, K)], srcv[b],
                                  sem_i[b]).wait()
            pltpu.make_async_copy(dst_hbm.at[pl.ds(0, K)], dstv[b],
                                  sem_i[b]).wait()
            if scaled:
                pltpu.make_async_copy(ew_hbm.at[pl.ds(0, K)], ewv[b],
                                      sem_i[b]).wait()

        def issue_gather(b):
            pltpu.async_copy(x_hbm.at[srcv[b]], rows[b], sem_g[b])

        def wait_gather(b):
            pltpu.make_async_copy(x_hbm.at[srcv[b]], rows[b], sem_g[b]).wait()

        def issue_scatter(b):
            pltpu.async_copy(rows[b], accum.at[dstv[b]], sem_s[b], add=True)

        def wait_scatter(b):
            pltpu.make_async_copy(rows[b], accum.at[dstv[b]], sem_s[b]).wait()

        def scale(b):
            for j in range(K // 16):
                s16 = srcv[b][pl.ds(j * 16, 16)]
                d16 = dstv[b][pl.ds(j * 16, 16)]
                e16 = ewv[b][pl.ds(j * 16, 16)]
                n16 = (plsc.load_gather(dinvt, [s16]) * e16 *
                       plsc.load_gather(dinvt, [d16]))
                for l in range(16):
                    e = j * 16 + l
                    s = n16[l]
                    for cc in range(C // 16):
                        sl = pl.ds(cc * 16, 16)
                        rows[b][e, sl] = rows[b][e, sl] * s

        def handle(c, s0, s1, s2, skip_ws=False, prep_idx=True,
                   prep_gather=True):
            if prep_idx:
                if not skip_ws:
                    wait_scatter(s2)
                issue_idx(s2, c + 2)
            if prep_gather:
                wait_idx(s1)
                issue_gather(s1)
            wait_gather(s0)
            if scaled:
                scale(s0)
            issue_scatter(s0)

        issue_idx(0, 0)
        issue_idx(1, 1)
        wait_idx(0)
        issue_gather(0)
        handle(0, 0, 1, 2, skip_ws=True)
        handle(1, 1, 2, 0)
        handle(2, 2, 0, 1)

        def triple(i3, carry):
            handle(3 * i3, 0, 1, 2)
            handle(3 * i3 + 1, 1, 2, 0)
            handle(3 * i3 + 2, 2, 0, 1)
            return carry

        lax.fori_loop(1, (NCHUNK - 2) // 3, triple, 0)
        handle(NCHUNK - 2, 0, 1, 2, prep_idx=False)
        handle(NCHUNK - 1, 1, 2, 0, prep_idx=False, prep_gather=False)
        wait_scatter(2)
        wait_scatter(0)
        wait_scatter(1)
        plsc.subcore_barrier()
        pltpu.sync_copy(accum.at[pl.ds(tile * RPT, RPT)],
                        z_out.at[cid, pl.ds(tile * RPT, RPT)])

    return pl.kernel(
        body,
        out_type=jax.ShapeDtypeStruct((NC, NP, C), _f32),
        scratch_types=scratch,
        **_MESH,
    )


_agg_scaled = _make_agg(True)
_agg_plain = _make_agg(False)


_RB = 2000


def _h_body(z_ref, x_ref, deg_ref, w1_ref, b1_ref, h_ref, y2_ref):
    dg = deg_ref[0] + deg_ref[1]
    dw = lax.rsqrt(dg[0] + 1.0)
    d1 = lax.rsqrt(dg[1] + 1.0)
    s1 = z_ref[0] + z_ref[1] + dw * dw * x_ref[...]
    h = jnp.dot(s1, w1_ref[...], preferred_element_type=_f32) + b1_ref[...]
    h = jnp.maximum(h, 0.0)
    h_ref[...] = h
    y2_ref[...] = d1 * h


_h_call = pl.pallas_call(
    _h_body,
    grid=(N // _RB,),
    in_specs=[
        pl.BlockSpec((NC, _RB, C), lambda i: (0, i, 0)),
        pl.BlockSpec((_RB, C), lambda i: (i, 0)),
        pl.BlockSpec((NC, 2, _RB, 1), lambda i: (0, 0, i, 0)),
        pl.BlockSpec((C, C), lambda i: (0, 0)),
        pl.BlockSpec((1, C), lambda i: (0, 0)),
    ],
    out_specs=[
        pl.BlockSpec((_RB, C), lambda i: (i, 0)),
        pl.BlockSpec((_RB, C), lambda i: (i, 0)),
    ],
    out_shape=[
        jax.ShapeDtypeStruct((N, C), _f32),
        jax.ShapeDtypeStruct((N, C), _f32),
    ],
)


def _out_body(z_ref, h_ref, deg_ref, wmu_ref, bmu_ref, wls_ref, bls_ref,
              mu_ref, ls_ref):
    dg = deg_ref[0] + deg_ref[1]
    d1 = lax.rsqrt(dg[1] + 1.0)
    g = d1 * (z_ref[0] + z_ref[1]) + d1 * d1 * h_ref[...]
    mu_ref[...] = jnp.dot(g, wmu_ref[...], preferred_element_type=_f32) + bmu_ref[...]
    ls_ref[...] = jnp.dot(g, wls_ref[...], preferred_element_type=_f32) + bls_ref[...]


_out_call = pl.pallas_call(
    _out_body,
    grid=(N // _RB,),
    in_specs=[
        pl.BlockSpec((NC, _RB, C), lambda i: (0, i, 0)),
        pl.BlockSpec((_RB, C), lambda i: (i, 0)),
        pl.BlockSpec((NC, 2, _RB, 1), lambda i: (0, 0, i, 0)),
        pl.BlockSpec((C, OC), lambda i: (0, 0)),
        pl.BlockSpec((1, OC), lambda i: (0, 0)),
        pl.BlockSpec((C, OC), lambda i: (0, 0)),
        pl.BlockSpec((1, OC), lambda i: (0, 0)),
    ],
    out_specs=[
        pl.BlockSpec((_RB, OC), lambda i: (i, 0)),
        pl.BlockSpec((_RB, OC), lambda i: (i, 0)),
    ],
    out_shape=[
        jax.ShapeDtypeStruct((N, OC), _f32),
        jax.ShapeDtypeStruct((N, OC), _f32),
    ],
)


def kernel(X, edge_index, edge_weight, W1, b1, Wmu, bmu, Wls, bls):
    src = edge_index[0].astype(_i32)
    dst = edge_index[1].astype(_i32)
    ew = edge_weight.astype(_f32)
    zeros128 = jnp.zeros((NP, C), _f32)

    deg = _deg_kernel(dst, ew)
    degc = deg.reshape(NC, 2, NP, 1)
    z1 = _agg_scaled(src, dst, ew, deg, X, zeros128)
    h, y2 = _h_call(z1, X, degc, W1, b1.reshape(1, C))
    z2 = _agg_plain(src, dst, y2, zeros128)
    mu, ls = _out_call(z2, h, degc, Wmu, bmu.reshape(1, OC),
                       Wls, bls.reshape(1, OC))
    return (mu, ls)
